# baseline (speedup 1.0000x reference)
import functools

import jax
import jax.numpy as jnp
from jax import lax
from jax.experimental import pallas as pl
from jax.experimental.pallas import tpu as pltpu
from jax.experimental.pallas import tpu_sc as plsc

NC = 2
NS = 16
LB = 64
DEGW = 16

_F32 = jnp.float32
_HIGH = jax.lax.Precision.HIGHEST


def _dotT(a, b):
    return lax.dot_general(a, b, (((1,), (1,)), ((), ())),
                           precision=_HIGH, preferred_element_type=_F32)


def _zero_block(buf, width):
    zv = jnp.zeros((16,), _F32)

    def body(i, carry):
        for k in range(width // 16):
            buf[i, pl.ds(k * 16, 16)] = zv
        return carry

    lax.fori_loop(0, buf.shape[0], body, 0)


def _zero_spmem_rows(shared, srcz, row0, nrows):
    rz = srcz.shape[0]
    full, tail = nrows // rz, nrows % rz
    for k in range(full):
        pltpu.sync_copy(srcz,
                        shared.at[pl.ds(pl.multiple_of(row0 + k * rz, 8), rz)])
    if tail:
        pltpu.sync_copy(srcz.at[pl.ds(0, tail)],
                        shared.at[pl.ds(pl.multiple_of(row0 + full * rz, 8), tail)])


@functools.lru_cache(maxsize=None)
def _make_deg_kernel(NP, NBLK):
    rpt = NP // NS
    mesh = plsc.VectorSubcoreMesh(core_axis_name="c", subcore_axis_name="s",
                                  num_cores=NC, num_subcores=NS)

    @functools.partial(
        pl.kernel,
        mesh=mesh,
        out_type=jax.ShapeDtypeStruct((NC * NP, DEGW), _F32),
        scratch_types=[
            pltpu.VMEM((NBLK, LB), jnp.int32),
            pltpu.VMEM((LB, DEGW), _F32),
            pltpu.VMEM((LB, DEGW), _F32),
            pltpu.VMEM_SHARED((NP, DEGW), _F32),
        ],
    )
    def deg_kernel(c2d_hbm, out_hbm, c_v, ones_v, zeros_v, hist_sh):
        cid = lax.axis_index("c")
        sid = lax.axis_index("s")
        wid = cid * NS + sid

        ov = jnp.full((16,), 1.0, _F32)

        def init(i, carry):
            ones_v[i] = ov
            return carry

        lax.fori_loop(0, ones_v.shape[0], init, 0)
        _zero_block(zeros_v, DEGW)
        _zero_spmem_rows(hist_sh, zeros_v, sid * rpt, rpt)

        pltpu.sync_copy(c2d_hbm.at[pl.ds(pl.multiple_of(wid * NBLK, 8), NBLK)], c_v)
        plsc.subcore_barrier()

        def body(j, carry):
            pltpu.sync_copy(ones_v, hist_sh.at[c_v.at[j]], add=True)
            return carry

        lax.fori_loop(0, NBLK, body, 0)
        plsc.subcore_barrier()
        pltpu.sync_copy(hist_sh.at[pl.ds(pl.multiple_of(sid * rpt, 8), rpt)],
                        out_hbm.at[pl.ds(pl.multiple_of(cid * NP + sid * rpt, 8), rpt)])

    return deg_kernel


@functools.lru_cache(maxsize=None)
def _make_scatter_kernel(NP, F, NB0, NB1, CH):
    rpt = NP // NS
    IC = 16
    assert NB0 % IC == 0 and NB1 % IC == 0
    mesh = plsc.VectorSubcoreMesh(core_axis_name="c", subcore_axis_name="s",
                                  num_cores=NC, num_subcores=NS)

    @functools.partial(
        pl.kernel,
        mesh=mesh,
        out_type=jax.ShapeDtypeStruct((NC * NP, F), _F32),
        scratch_types=[
            pltpu.VMEM((IC, LB), jnp.int32),
            pltpu.VMEM((IC, LB), jnp.int32),
            pltpu.VMEM((CH, LB, F), _F32),
            pltpu.VMEM_SHARED((NP, F), _F32),
            pltpu.SemaphoreType.DMA,
        ],
    )
    def scatter_kernel(u_hbm, r2d_hbm, c2d_hbm, out_hbm,
                       r_v, c_v, rows_v, acc_sh, sem):
        cid = lax.axis_index("c")
        sid = lax.axis_index("s")
        base = jnp.where(cid == 0, sid * NB0, NS * NB0 + sid * NB1)
        nchunks = jnp.where(cid == 0, NB0 // IC, NB1 // IC)

        _zero_block(rows_v.at[0], F)
        _zero_spmem_rows(acc_sh, rows_v.at[0], sid * rpt, rpt)
        plsc.subcore_barrier()

        def chunk(ci, carry):
            off = pl.multiple_of(base + ci * IC, 8)
            pltpu.sync_copy(r2d_hbm.at[pl.ds(off, IC)], r_v)
            pltpu.sync_copy(c2d_hbm.at[pl.ds(off, IC)], c_v)

            def fire(j):
                return pltpu.async_copy(
                    u_hbm.at[r_v.at[j]], rows_v.at[j % CH], sem)

            descs = [None] * IC
            for j in range(min(CH, IC)):
                descs[j] = fire(j)
            for j in range(IC):
                descs[j].wait()
                if j + CH < IC:
                    descs[j + CH] = fire(j + CH)
                pltpu.sync_copy(rows_v.at[j % CH], acc_sh.at[c_v.at[j]],
                                add=True)
            return carry

        lax.fori_loop(0, nchunks, chunk, 0)
        plsc.subcore_barrier()
        pltpu.sync_copy(acc_sh.at[pl.ds(pl.multiple_of(sid * rpt, 8), rpt)],
                        out_hbm.at[pl.ds(pl.multiple_of(cid * NP + sid * rpt, 8), rpt)])

    return scatter_kernel


def _tc_prep_body(N, NP, x_ref, w1_ref, degp_ref, u_ref, dinv_ref):
    deg = degp_ref[0:N, 0:1] + degp_ref[NP:NP + N, 0:1] + 1.0
    dinv = lax.rsqrt(deg)
    dinv_ref[...] = dinv
    u = _dotT(x_ref[...], w1_ref[...]) * dinv
    u_ref[0:N, :] = u
    u_ref[N:NP, :] = jnp.zeros((NP - N, u.shape[1]), _F32)


def _tc_combine_body(N, NP, accp_ref, u_ref, dinv_ref,
                     b_ref, g_ref, be_ref, h_ref):
    dinv = dinv_ref[...]
    s = u_ref[0:N, :] + accp_ref[0:N, :] + accp_ref[NP:NP + N, :]
    t = s * dinv + b_ref[...]
    t = jnp.maximum(t, 0.0)
    mu = jnp.mean(t, axis=0, keepdims=True)
    var = jnp.mean(t * t, axis=0, keepdims=True) - mu * mu
    h_ref[...] = (t - mu) * lax.rsqrt(var + 1e-5) * g_ref[...] + be_ref[...]


def _tc_pool_next_body(N, NP, G, h_ref, dinv_ref, batch_ref, w_ref,
                       un_ref, pool_ref):
    h = h_ref[...]
    seg = lax.broadcasted_iota(jnp.int32, (N, G), 1)
    onehot = (batch_ref[...] == seg).astype(_F32)
    pool_ref[...] = lax.dot_general(onehot, h, (((0,), (0,)), ((), ())),
                                    precision=_HIGH,
                                    preferred_element_type=_F32)
    un = _dotT(h, w_ref[...]) * dinv_ref[...]
    un_ref[0:N, :] = un
    un_ref[N:NP, :] = jnp.zeros((NP - N, un.shape[1]), _F32)


def kernel(x, edge_index, batch, W1, b1, g1, be1, W2, b2, g2, be2,
           W3, b3, g3, be3):
    N, F = x.shape
    E = edge_index.shape[1]
    G = 64
    H = W1.shape[0]

    CH = 5
    NP = ((N + 127) // 128) * 128
    if NP == N:
        NP += 128
    epb = NC * NS * LB * 16
    E_pad = ((E + epb - 1) // epb) * epb
    NBLK = E_pad // (NC * NS * LB)

    r = jnp.concatenate([edge_index[0],
                         jnp.full((E_pad - E,), N, jnp.int32)]).reshape(-1, LB)
    c = jnp.concatenate([edge_index[1],
                         jnp.full((E_pad - E,), N, jnp.int32)]).reshape(-1, LB)
    batch2d = batch.reshape(N, 1)

    NBLK_PAIR = E_pad // (NS * LB)
    NB0 = (NBLK_PAIR // 5) // 16 * 16
    NB1 = NBLK_PAIR - NB0
    deg_k = _make_deg_kernel(NP, NBLK)
    scat_k = _make_scatter_kernel(NP, H, NB0, NB1, CH)

    degp = deg_k(c)

    tc_prep = pl.pallas_call(
        functools.partial(_tc_prep_body, N, NP),
        out_shape=(jax.ShapeDtypeStruct((NP, H), _F32),
                   jax.ShapeDtypeStruct((N, 1), _F32)),
    )
    u1, dinv = tc_prep(x, W1, degp)

    tc_combine = pl.pallas_call(
        functools.partial(_tc_combine_body, N, NP),
        out_shape=jax.ShapeDtypeStruct((N, H), _F32))
    tc_pool_next = pl.pallas_call(
        functools.partial(_tc_pool_next_body, N, NP, G),
        out_shape=(jax.ShapeDtypeStruct((NP, H), _F32),
                   jax.ShapeDtypeStruct((G, H), _F32)))

    bs = jnp.stack([b1, b2, b3]).reshape(3, 1, H)
    gs = jnp.stack([g1, g2, g3]).reshape(3, 1, H)
    bes = jnp.stack([be1, be2, be3]).reshape(3, 1, H)
    Ws = jnp.stack([W2, W3, W3])

    def step(carry, p):
        u_cur, _ = carry
        b_, g_, be_, W_ = p
        accp = scat_k(u_cur, r, c)
        h = tc_combine(accp, u_cur, dinv, b_, g_, be_)
        un, pool = tc_pool_next(h, dinv, batch2d, W_)
        return (un, h), pool

    h0 = jnp.zeros((N, H), _F32)
    (_, h3), pools = lax.scan(step, (u1, h0), (bs, gs, bes, Ws))

    global_rep = jnp.concatenate([pools[0], pools[1], pools[2]], axis=1)
    return (global_rep, h3)


# speedup vs baseline: 1.0671x; 1.0671x over previous
import functools

import jax
import jax.numpy as jnp
from jax import lax
from jax.experimental import pallas as pl
from jax.experimental.pallas import tpu as pltpu
from jax.experimental.pallas import tpu_sc as plsc

NC = 2
NS = 16
LB = 64
DEGW = 16

_F32 = jnp.float32
_HIGH = jax.lax.Precision.HIGHEST


def _dotT(a, b):
    return lax.dot_general(a, b, (((1,), (1,)), ((), ())),
                           precision=_HIGH, preferred_element_type=_F32)


def _zero_block(buf, width):
    zv = jnp.zeros((16,), _F32)

    def body(i, carry):
        for k in range(width // 16):
            buf[i, pl.ds(k * 16, 16)] = zv
        return carry

    lax.fori_loop(0, buf.shape[0], body, 0)


def _zero_spmem_rows(shared, srcz, row0, nrows):
    rz = srcz.shape[0]
    full, tail = nrows // rz, nrows % rz
    for k in range(full):
        pltpu.sync_copy(srcz,
                        shared.at[pl.ds(pl.multiple_of(row0 + k * rz, 8), rz)])
    if tail:
        pltpu.sync_copy(srcz.at[pl.ds(0, tail)],
                        shared.at[pl.ds(pl.multiple_of(row0 + full * rz, 8), tail)])


@functools.lru_cache(maxsize=None)
def _make_deg_kernel(NP, NBLK):
    rpt = NP // NS
    mesh = plsc.VectorSubcoreMesh(core_axis_name="c", subcore_axis_name="s",
                                  num_cores=NC, num_subcores=NS)

    @functools.partial(
        pl.kernel,
        mesh=mesh,
        out_type=jax.ShapeDtypeStruct((NC * NP, DEGW), _F32),
        scratch_types=[
            pltpu.VMEM((NBLK, LB), jnp.int32),
            pltpu.VMEM((LB, DEGW), _F32),
            pltpu.VMEM((LB, DEGW), _F32),
            pltpu.VMEM_SHARED((NP, DEGW), _F32),
        ],
    )
    def deg_kernel(c2d_hbm, out_hbm, c_v, ones_v, zeros_v, hist_sh):
        cid = lax.axis_index("c")
        sid = lax.axis_index("s")
        wid = cid * NS + sid

        ov = jnp.full((16,), 1.0, _F32)

        def init(i, carry):
            ones_v[i] = ov
            return carry

        lax.fori_loop(0, ones_v.shape[0], init, 0)
        _zero_block(zeros_v, DEGW)
        _zero_spmem_rows(hist_sh, zeros_v, sid * rpt, rpt)

        pltpu.sync_copy(c2d_hbm.at[pl.ds(pl.multiple_of(wid * NBLK, 8), NBLK)], c_v)
        plsc.subcore_barrier()

        def body(j, carry):
            pltpu.sync_copy(ones_v, hist_sh.at[c_v.at[j]], add=True)
            return carry

        lax.fori_loop(0, NBLK, body, 0)
        plsc.subcore_barrier()
        pltpu.sync_copy(hist_sh.at[pl.ds(pl.multiple_of(sid * rpt, 8), rpt)],
                        out_hbm.at[pl.ds(pl.multiple_of(cid * NP + sid * rpt, 8), rpt)])

    return deg_kernel


@functools.lru_cache(maxsize=None)
def _make_scatter_kernel(NP, F, NB0, NB1, CH):
    rpt = NP // NS
    IC = 16
    assert NB0 % IC == 0 and NB1 % IC == 0
    mesh = plsc.VectorSubcoreMesh(core_axis_name="c", subcore_axis_name="s",
                                  num_cores=NC, num_subcores=NS)

    @functools.partial(
        pl.kernel,
        mesh=mesh,
        out_type=jax.ShapeDtypeStruct((NC * NP, F), _F32),
        scratch_types=[
            pltpu.VMEM((IC, LB), jnp.int32),
            pltpu.VMEM((IC, LB), jnp.int32),
            pltpu.VMEM((CH, LB, F), _F32),
            pltpu.VMEM_SHARED((NP, F), _F32),
            pltpu.SemaphoreType.DMA,
        ],
    )
    def scatter_kernel(u_hbm, r2d_hbm, c2d_hbm, out_hbm,
                       r_v, c_v, rows_v, acc_sh, sem):
        cid = lax.axis_index("c")
        sid = lax.axis_index("s")
        assert NB0 == NB1
        wid = cid * NS + sid

        _zero_block(rows_v.at[0], F)
        _zero_spmem_rows(acc_sh, rows_v.at[0], sid * rpt, rpt)
        plsc.subcore_barrier()

        def chunk(ci, carry):
            off = pl.multiple_of(wid * NB0 + ci * IC, 8)
            pltpu.sync_copy(r2d_hbm.at[pl.ds(off, IC)], r_v)
            pltpu.sync_copy(c2d_hbm.at[pl.ds(off, IC)], c_v)

            def fire(j):
                return pltpu.async_copy(
                    u_hbm.at[r_v.at[j]], rows_v.at[j % CH], sem)

            descs = [None] * IC
            for j in range(min(CH, IC)):
                descs[j] = fire(j)
            for j in range(IC):
                descs[j].wait()
                if j + CH < IC:
                    descs[j + CH] = fire(j + CH)
                pltpu.sync_copy(rows_v.at[j % CH], acc_sh.at[c_v.at[j]],
                                add=True)
            return carry

        lax.fori_loop(0, NB0 // IC, chunk, 0)
        plsc.subcore_barrier()
        pltpu.sync_copy(acc_sh.at[pl.ds(pl.multiple_of(sid * rpt, 8), rpt)],
                        out_hbm.at[pl.ds(pl.multiple_of(cid * NP + sid * rpt, 8), rpt)])

    return scatter_kernel


def _tc_prep_body(N, NP, x_ref, w1_ref, degp_ref, u_ref, dinv_ref):
    deg = degp_ref[0:N, 0:1] + degp_ref[NP:NP + N, 0:1] + 1.0
    dinv = lax.rsqrt(deg)
    dinv_ref[...] = dinv
    u = _dotT(x_ref[...], w1_ref[...]) * dinv
    u_ref[0:N, :] = u
    u_ref[N:NP, :] = jnp.zeros((NP - N, u.shape[1]), _F32)


def _tc_combine_body(N, NP, accp_ref, u_ref, dinv_ref,
                     b_ref, g_ref, be_ref, h_ref):
    dinv = dinv_ref[...]
    s = u_ref[0:N, :] + accp_ref[0:N, :] + accp_ref[NP:NP + N, :]
    t = s * dinv + b_ref[...]
    t = jnp.maximum(t, 0.0)
    mu = jnp.mean(t, axis=0, keepdims=True)
    var = jnp.mean(t * t, axis=0, keepdims=True) - mu * mu
    h_ref[...] = (t - mu) * lax.rsqrt(var + 1e-5) * g_ref[...] + be_ref[...]


def _tc_pool_next_body(N, NP, G, h_ref, dinv_ref, batch_ref, w_ref,
                       un_ref, pool_ref):
    h = h_ref[...]
    seg = lax.broadcasted_iota(jnp.int32, (N, G), 1)
    onehot = (batch_ref[...] == seg).astype(_F32)
    pool_ref[...] = lax.dot_general(onehot, h, (((0,), (0,)), ((), ())),
                                    precision=_HIGH,
                                    preferred_element_type=_F32)
    un = _dotT(h, w_ref[...]) * dinv_ref[...]
    un_ref[0:N, :] = un
    un_ref[N:NP, :] = jnp.zeros((NP - N, un.shape[1]), _F32)


def kernel(x, edge_index, batch, W1, b1, g1, be1, W2, b2, g2, be2,
           W3, b3, g3, be3):
    N, F = x.shape
    E = edge_index.shape[1]
    G = 64
    H = W1.shape[0]

    CH = 5
    NP = ((N + 127) // 128) * 128
    if NP == N:
        NP += 128
    epb = NC * NS * LB * 16
    E_pad = ((E + epb - 1) // epb) * epb
    NBLK = E_pad // (NC * NS * LB)

    r = jnp.concatenate([edge_index[0],
                         jnp.full((E_pad - E,), N, jnp.int32)]).reshape(-1, LB)
    c = jnp.concatenate([edge_index[1],
                         jnp.full((E_pad - E,), N, jnp.int32)]).reshape(-1, LB)
    batch2d = batch.reshape(N, 1)

    NBLK_PAIR = E_pad // (NS * LB)
    NB0 = NBLK_PAIR // 2
    NB1 = NBLK_PAIR - NB0
    deg_k = _make_deg_kernel(NP, NBLK)
    scat_k = _make_scatter_kernel(NP, H, NB0, NB1, CH)

    degp = deg_k(c)

    tc_prep = pl.pallas_call(
        functools.partial(_tc_prep_body, N, NP),
        out_shape=(jax.ShapeDtypeStruct((NP, H), _F32),
                   jax.ShapeDtypeStruct((N, 1), _F32)),
    )
    u1, dinv = tc_prep(x, W1, degp)

    tc_combine = pl.pallas_call(
        functools.partial(_tc_combine_body, N, NP),
        out_shape=jax.ShapeDtypeStruct((N, H), _F32))
    tc_pool_next = pl.pallas_call(
        functools.partial(_tc_pool_next_body, N, NP, G),
        out_shape=(jax.ShapeDtypeStruct((NP, H), _F32),
                   jax.ShapeDtypeStruct((G, H), _F32)))

    bs = jnp.stack([b1, b2, b3]).reshape(3, 1, H)
    gs = jnp.stack([g1, g2, g3]).reshape(3, 1, H)
    bes = jnp.stack([be1, be2, be3]).reshape(3, 1, H)
    Ws = jnp.stack([W2, W3, W3])

    def step(carry, p):
        u_cur, _ = carry
        b_, g_, be_, W_ = p
        accp = scat_k(u_cur, r, c)
        h = tc_combine(accp, u_cur, dinv, b_, g_, be_)
        un, pool = tc_pool_next(h, dinv, batch2d, W_)
        return (un, h), pool

    h0 = jnp.zeros((N, H), _F32)
    (_, h3), pools = lax.scan(step, (u1, h0), (bs, gs, bes, Ws))

    global_rep = jnp.concatenate([pools[0], pools[1], pools[2]], axis=1)
    return (global_rep, h3)
